# baseline (speedup 1.0000x reference)
import functools

import jax
import jax.numpy as jnp
from jax import lax
from jax.experimental import pallas as pl
from jax.experimental.pallas import tpu as pltpu
from jax.experimental.pallas import tpu_sc as plsc

from antenv.accelerators import mock_tpu
from axiom.mock_tpu import make_compilable_single_device_mesh

_BATCH = 16384
_EMBED = 64
_NW = 32
_BPW = _BATCH // _NW
_CHUNK = 32
_NCHUNK = _BPW // _CHUNK


def _make_gather():
    mesh = plsc.VectorSubcoreMesh(core_axis_name="c", subcore_axis_name="s")

    @functools.partial(
        pl.kernel,
        mesh=mesh,
        out_type=jax.ShapeDtypeStruct((_BATCH, _EMBED), jnp.float32),
        scratch_types=[
            pltpu.VMEM((4, 128), jnp.int32),
            pltpu.VMEM((3 * _CHUNK, 8, _EMBED), jnp.float32),
            pltpu.VMEM((2 * _CHUNK, _EMBED), jnp.float32),
            pltpu.SemaphoreType.DMA,
            pltpu.SemaphoreType.DMA,
        ],
        compiler_params=pltpu.CompilerParams(
            use_tc_tiling_on_sc=True, needs_layout_passes=False
        ),
    )
    def gather_kernel(idx_hbm, table_hbm, out_hbm, xv, bands2, stage, sem, osem):

        wid = lax.axis_index("s") * 2 + lax.axis_index("c")
        base = wid * _BPW
        for j in range(4):
            pltpu.sync_copy(idx_hbm.at[pl.ds(base + j * 128, 128)], xv.at[j])

        def enqueue(c, buf):
            blks = [
                xv[c >> 2, pl.ds((c & 3) * _CHUNK + t * 16, 16)]
                for t in range(_CHUNK // 16)
            ]
            for k in range(_CHUNK):
                v = blks[k // 16][k % 16]
                pltpu.async_copy(
                    table_hbm.at[pl.ds(v >> 3, 1)],
                    bands2.at[pl.ds(buf * _CHUNK + k, 1)],
                    sem,
                )

        enqueue(0, 0)
        enqueue(1, 1)

        def chunk_body(g, _):
            nxt = g + 2

            @pl.when(nxt < _NCHUNK)
            def _():
                enqueue(nxt, nxt - (nxt // 3) * 3)

            gb = g - (g // 3) * 3
            pltpu.make_async_copy(
                table_hbm.at[pl.ds(0, _CHUNK)],
                bands2.at[pl.ds(gb * _CHUNK, _CHUNK)],
                sem,
            ).wait()

            @pl.when(g >= 2)
            def _():
                pltpu.make_async_copy(
                    stage.at[pl.ds((g & 1) * _CHUNK, _CHUNK)],
                    out_hbm.at[pl.ds(base, _CHUNK)],
                    osem,
                ).wait()

            blks = [
                xv[g >> 2, pl.ds((g & 3) * _CHUNK + t * 16, 16)]
                for t in range(_CHUNK // 16)
            ]
            for k in range(_CHUNK):
                v = blks[k // 16][k % 16]
                b = gb * _CHUNK + k
                for eb in range(_EMBED // 16):
                    stage[(g & 1) * _CHUNK + k, pl.ds(eb * 16, 16)] = bands2[b, v & 7, pl.ds(eb * 16, 16)]
            pltpu.async_copy(
                stage.at[pl.ds((g & 1) * _CHUNK, _CHUNK)],
                out_hbm.at[pl.ds(base + g * _CHUNK, _CHUNK)],
                osem,
            )
            return ()

        lax.fori_loop(0, _NCHUNK, chunk_body, ())
        for _t in range(2):
            pltpu.make_async_copy(
                stage.at[pl.ds(_t * _CHUNK, _CHUNK)],
                out_hbm.at[pl.ds(base, _CHUNK)],
                osem,
            ).wait()

    return gather_kernel

_gather = _make_gather()


@jax.jit
def kernel(X, embedding):
    t3 = jnp.reshape(embedding, (125000, 8, _EMBED))
    return _gather(X.astype(jnp.int32), t3)


# speedup vs baseline: 1.0008x; 1.0008x over previous
import functools

import jax
import jax.numpy as jnp
from jax import lax
from jax.experimental import pallas as pl
from jax.experimental.pallas import tpu as pltpu
from jax.experimental.pallas import tpu_sc as plsc

from antenv.accelerators import mock_tpu
from axiom.mock_tpu import make_compilable_single_device_mesh

_BATCH = 16384
_EMBED = 64
_NW = 32
_BPW = _BATCH // _NW
_CHUNK = 32
_NCHUNK = _BPW // _CHUNK


def _make_gather():
    mesh = plsc.VectorSubcoreMesh(core_axis_name="c", subcore_axis_name="s")

    @functools.partial(
        pl.kernel,
        mesh=mesh,
        out_type=jax.ShapeDtypeStruct((_BATCH, _EMBED), jnp.float32),
        scratch_types=[
            pltpu.VMEM((4, 128), jnp.int32),
            pltpu.VMEM((2 * _CHUNK, 8, _EMBED), jnp.float32),
            pltpu.VMEM((2 * _CHUNK, _EMBED), jnp.float32),
            pltpu.SemaphoreType.DMA,
            pltpu.SemaphoreType.DMA,
        ],
        compiler_params=pltpu.CompilerParams(
            use_tc_tiling_on_sc=True, needs_layout_passes=False
        ),
    )
    def gather_kernel(idx_hbm, table_hbm, out_hbm, xv, bands2, stage, sem, osem):

        wid = lax.axis_index("s") * 2 + lax.axis_index("c")
        base = wid * _BPW
        for j in range(4):
            pltpu.sync_copy(idx_hbm.at[pl.ds(base + j * 128, 128)], xv.at[j])

        def enqueue(c, buf):
            blks = [
                xv[c >> 2, pl.ds((c & 3) * _CHUNK + t * 16, 16)]
                for t in range(_CHUNK // 16)
            ]
            for k in range(_CHUNK):
                v = blks[k // 16][k % 16]
                pltpu.async_copy(
                    table_hbm.at[pl.ds(v >> 3, 1)],
                    bands2.at[pl.ds(buf * _CHUNK + k, 1)],
                    sem,
                )

        enqueue(0, 0)

        def chunk_body(g, _):
            nxt = g + 1

            @pl.when(nxt < _NCHUNK)
            def _():
                enqueue(nxt, nxt & 1)

            pltpu.make_async_copy(
                table_hbm.at[pl.ds(0, _CHUNK)],
                bands2.at[pl.ds((g & 1) * _CHUNK, _CHUNK)],
                sem,
            ).wait()

            @pl.when(g >= 2)
            def _():
                pltpu.make_async_copy(
                    stage.at[pl.ds((g & 1) * _CHUNK, _CHUNK)],
                    out_hbm.at[pl.ds(base, _CHUNK)],
                    osem,
                ).wait()

            blks = [
                xv[g >> 2, pl.ds((g & 3) * _CHUNK + t * 16, 16)]
                for t in range(_CHUNK // 16)
            ]
            for k in range(_CHUNK):
                v = blks[k // 16][k % 16]
                b = (g & 1) * _CHUNK + k
                for eb in range(_EMBED // 16):
                    stage[(g & 1) * _CHUNK + k, pl.ds(eb * 16, 16)] = bands2[b, v & 7, pl.ds(eb * 16, 16)]
            pltpu.async_copy(
                stage.at[pl.ds((g & 1) * _CHUNK, _CHUNK)],
                out_hbm.at[pl.ds(base + g * _CHUNK, _CHUNK)],
                osem,
            )
            return ()

        lax.fori_loop(0, _NCHUNK, chunk_body, ())
        for _t in range(2):
            pltpu.make_async_copy(
                stage.at[pl.ds(_t * _CHUNK, _CHUNK)],
                out_hbm.at[pl.ds(base, _CHUNK)],
                osem,
            ).wait()

    return gather_kernel

_gather = _make_gather()


@jax.jit
def kernel(X, embedding):
    t3 = jnp.reshape(embedding, (125000, 8, _EMBED))
    return _gather(X.astype(jnp.int32), t3)


# speedup vs baseline: 1.0027x; 1.0019x over previous
import functools

import jax
import jax.numpy as jnp
from jax import lax
from jax.experimental import pallas as pl
from jax.experimental.pallas import tpu as pltpu
from jax.experimental.pallas import tpu_sc as plsc

_BATCH = 16384
_EMBED = 64
_NW = 32
_BPW = _BATCH // _NW
_CHUNK = 32
_NCHUNK = _BPW // _CHUNK


def _make_gather():
    mesh = plsc.VectorSubcoreMesh(core_axis_name="c", subcore_axis_name="s")

    @functools.partial(
        pl.kernel,
        mesh=mesh,
        out_type=jax.ShapeDtypeStruct((_BATCH, _EMBED), jnp.float32),
        scratch_types=[
            pltpu.VMEM((4, 128), jnp.int32),
            pltpu.VMEM((2 * _CHUNK, 8, _EMBED), jnp.float32),
            pltpu.VMEM((2 * _CHUNK, _EMBED), jnp.float32),
            pltpu.SemaphoreType.DMA,
            pltpu.SemaphoreType.DMA,
        ],
        compiler_params=pltpu.CompilerParams(
            use_tc_tiling_on_sc=True, needs_layout_passes=False
        ),
    )
    def gather_kernel(idx_hbm, table_hbm, out_hbm, xv, bands2, stage, sem, osem):

        wid = lax.axis_index("s") * 2 + lax.axis_index("c")
        base = wid * _BPW
        for j in range(4):
            pltpu.sync_copy(idx_hbm.at[pl.ds(base + j * 128, 128)], xv.at[j])

        def enqueue(c, buf):
            blks = [
                xv[c >> 2, pl.ds((c & 3) * _CHUNK + t * 16, 16)]
                for t in range(_CHUNK // 16)
            ]
            for k in range(_CHUNK):
                v = blks[k // 16][k % 16]
                pltpu.async_copy(
                    table_hbm.at[pl.ds(v >> 3, 1)],
                    bands2.at[pl.ds(buf * _CHUNK + k, 1)],
                    sem,
                )

        enqueue(0, 0)

        def chunk_body(g, _):
            nxt = g + 1

            @pl.when(nxt < _NCHUNK)
            def _():
                enqueue(nxt, nxt & 1)

            pltpu.make_async_copy(
                table_hbm.at[pl.ds(0, _CHUNK)],
                bands2.at[pl.ds((g & 1) * _CHUNK, _CHUNK)],
                sem,
            ).wait()

            @pl.when(g >= 2)
            def _():
                pltpu.make_async_copy(
                    stage.at[pl.ds((g & 1) * _CHUNK, _CHUNK)],
                    out_hbm.at[pl.ds(base, _CHUNK)],
                    osem,
                ).wait()

            blks = [
                xv[g >> 2, pl.ds((g & 3) * _CHUNK + t * 16, 16)]
                for t in range(_CHUNK // 16)
            ]
            for k in range(_CHUNK):
                v = blks[k // 16][k % 16]
                b = (g & 1) * _CHUNK + k
                for eb in range(_EMBED // 16):
                    stage[(g & 1) * _CHUNK + k, pl.ds(eb * 16, 16)] = bands2[b, v & 7, pl.ds(eb * 16, 16)]
            pltpu.async_copy(
                stage.at[pl.ds((g & 1) * _CHUNK, _CHUNK)],
                out_hbm.at[pl.ds(base + g * _CHUNK, _CHUNK)],
                osem,
            )
            return ()

        lax.fori_loop(0, _NCHUNK, chunk_body, ())
        for _t in range(2):
            pltpu.make_async_copy(
                stage.at[pl.ds(_t * _CHUNK, _CHUNK)],
                out_hbm.at[pl.ds(base, _CHUNK)],
                osem,
            ).wait()

    return gather_kernel

_gather = _make_gather()


@jax.jit
def kernel(X, embedding):
    t3 = jnp.reshape(embedding, (125000, 8, _EMBED))
    return _gather(X.astype(jnp.int32), t3)
